# baseline (speedup 1.0000x reference)
import functools

import jax
import jax.numpy as jnp
from jax.experimental import pallas as pl

EPS = 1e-6


def _dot(a, b):
    return jnp.dot(a, b, preferred_element_type=jnp.float32)


def _norm_mm_body(x_ref, nw_ref, w_ref, o_ref):
    xb = x_ref[...]
    v = jnp.mean(xb * xb, axis=1, keepdims=True)
    normed = xb * jax.lax.rsqrt(v + EPS) * nw_ref[...]
    o_ref[...] = _dot(normed, w_ref[...])


def _norm_mm(x, nw, w, bm=512, bn=1024):
    T, D = x.shape
    N = w.shape[1]
    return pl.pallas_call(
        _norm_mm_body,
        grid=(T // bm, N // bn),
        in_specs=[
            pl.BlockSpec((bm, D), lambda i, j: (i, 0)),
            pl.BlockSpec((1, D), lambda i, j: (0, 0)),
            pl.BlockSpec((D, bn), lambda i, j: (0, j)),
        ],
        out_specs=pl.BlockSpec((bm, bn), lambda i, j: (i, j)),
        out_shape=jax.ShapeDtypeStruct((T, N), jnp.float32),
    )(x, nw.reshape(1, D), w)


def _attn_body(q_ref, kt_ref, v_ref, o_ref, *, bq, T, scale):
    qb = pl.program_id(1)
    s = _dot(q_ref[0], kt_ref[0]) * scale
    col = jax.lax.broadcasted_iota(jnp.int32, (bq, T), 1)
    row = qb * bq + jax.lax.broadcasted_iota(jnp.int32, (bq, T), 0)
    s = jnp.where(col <= row, s, jnp.float32(-1e9))
    m = jnp.max(s, axis=1, keepdims=True)
    p = jnp.exp(s - m)
    a = p / jnp.sum(p, axis=1, keepdims=True)
    o_ref[0] = _dot(a, v_ref[0])


def _attn(q, kt, v, bq=256):
    H, T, HD = q.shape
    body = functools.partial(_attn_body, bq=bq, T=T,
                             scale=1.0 / (HD ** 0.5))
    return pl.pallas_call(
        body,
        grid=(H, T // bq),
        in_specs=[
            pl.BlockSpec((1, bq, HD), lambda h, i: (h, i, 0)),
            pl.BlockSpec((1, HD, T), lambda h, i: (h, 0, 0)),
            pl.BlockSpec((1, T, HD), lambda h, i: (h, 0, 0)),
        ],
        out_specs=pl.BlockSpec((1, bq, HD), lambda h, i: (h, i, 0)),
        out_shape=jax.ShapeDtypeStruct((H, T, HD), jnp.float32),
    )(q, kt, v)


def _out_norm_body(a_ref, wo_ref, x_ref, nw_ref, x1_ref, h2_ref):
    x1 = _dot(a_ref[...], wo_ref[...]) + x_ref[...]
    x1_ref[...] = x1
    v = jnp.mean(x1 * x1, axis=1, keepdims=True)
    h2_ref[...] = x1 * jax.lax.rsqrt(v + EPS) * nw_ref[...]


def _out_norm(attn2, wo, x, nw, bm=512):
    T, D = x.shape
    return pl.pallas_call(
        _out_norm_body,
        grid=(T // bm,),
        in_specs=[
            pl.BlockSpec((bm, D), lambda i: (i, 0)),
            pl.BlockSpec((D, D), lambda i: (0, 0)),
            pl.BlockSpec((bm, D), lambda i: (i, 0)),
            pl.BlockSpec((1, D), lambda i: (0, 0)),
        ],
        out_specs=[
            pl.BlockSpec((bm, D), lambda i: (i, 0)),
            pl.BlockSpec((bm, D), lambda i: (i, 0)),
        ],
        out_shape=[
            jax.ShapeDtypeStruct((T, D), jnp.float32),
            jax.ShapeDtypeStruct((T, D), jnp.float32),
        ],
    )(attn2, wo, x, nw.reshape(1, D))


def _router_body(h2_ref, rw_ref, w_ref, bal_ref, *, T, E):
    logits = _dot(h2_ref[...], rw_ref[...])
    lane = jax.lax.broadcasted_iota(jnp.int32, logits.shape, 1)
    valid = lane < E
    logits = jnp.where(valid, logits, jnp.float32(-1e30))
    m = jnp.max(logits, axis=1, keepdims=True)
    p = jnp.exp(logits - m)
    p = jnp.where(valid, p, 0.0)
    psum = jnp.sum(p, axis=1, keepdims=True)
    probs = p / psum
    pmin = jnp.min(jnp.where(valid, probs, jnp.float32(2.0)),
                   axis=1, keepdims=True)
    p1 = jnp.sum(jnp.where(lane == 1, probs, 0.0), axis=1, keepdims=True)
    p2 = jnp.sum(jnp.where(lane == 2, probs, 0.0), axis=1, keepdims=True)
    excl = jnp.where(p2 == pmin, 2, jnp.where(p1 == pmin, 1, 0))
    keep = valid & (lane != excl)
    denom = jnp.sum(probs, axis=1, keepdims=True) - pmin
    w_ref[...] = jnp.where(keep, probs, 0.0) / denom
    ft = jnp.sum(keep.astype(jnp.float32), axis=0, keepdims=True) / (T * 2.0)
    fp = jnp.sum(probs, axis=0, keepdims=True) / T
    bal = E * jnp.sum(ft * fp, axis=1, keepdims=True)
    bal_ref[...] = jnp.broadcast_to(bal, bal_ref.shape)


def _router(h2, rw_pad, E):
    T, D = h2.shape
    body = functools.partial(_router_body, T=T, E=E)
    return pl.pallas_call(
        body,
        grid=(1,),
        in_specs=[
            pl.BlockSpec((T, D), lambda i: (0, 0)),
            pl.BlockSpec((D, 128), lambda i: (0, 0)),
        ],
        out_specs=[
            pl.BlockSpec((T, 128), lambda i: (0, 0)),
            pl.BlockSpec((8, 128), lambda i: (0, 0)),
        ],
        out_shape=[
            jax.ShapeDtypeStruct((T, 128), jnp.float32),
            jax.ShapeDtypeStruct((8, 128), jnp.float32),
        ],
    )(h2, rw_pad)


def _moe_body(h2_ref, wts_ref, x1_ref, w1_ref, b1_ref, w2_ref, b2_ref, o_ref):
    e = pl.program_id(1)
    f = pl.program_id(2)
    lane = jax.lax.broadcasted_iota(jnp.int32, wts_ref.shape, 1)
    we = jnp.sum(jnp.where(lane == e, wts_ref[...], 0.0),
                 axis=1, keepdims=True)

    @pl.when((e == 0) & (f == 0))
    def _init():
        o_ref[...] = x1_ref[...]

    @pl.when(f == 0)
    def _bias2():
        o_ref[...] += we * b2_ref[0]

    h = jax.nn.gelu(_dot(h2_ref[...], w1_ref[0]) + b1_ref[0])
    o_ref[...] += _dot(we * h, w2_ref[0])


def _moe(h2, wts, x1, w1, b1, w2, b2, bm=256, bf=512):
    T, D = h2.shape
    E, _, F = w1.shape
    return pl.pallas_call(
        _moe_body,
        grid=(T // bm, E, F // bf),
        in_specs=[
            pl.BlockSpec((bm, D), lambda t, e, f: (t, 0)),
            pl.BlockSpec((bm, 128), lambda t, e, f: (t, 0)),
            pl.BlockSpec((bm, D), lambda t, e, f: (t, 0)),
            pl.BlockSpec((1, D, bf), lambda t, e, f: (e, 0, f)),
            pl.BlockSpec((1, 1, bf), lambda t, e, f: (e, 0, f)),
            pl.BlockSpec((1, bf, D), lambda t, e, f: (e, f, 0)),
            pl.BlockSpec((1, 1, D), lambda t, e, f: (e, 0, 0)),
        ],
        out_specs=pl.BlockSpec((bm, D), lambda t, e, f: (t, 0)),
        out_shape=jax.ShapeDtypeStruct((T, D), jnp.float32),
    )(h2, wts, x1, w1, b1.reshape(E, 1, F), w2, b2.reshape(E, 1, D))


def kernel(x, attn_norm_w, wq, wk, wv, wo, mlp_norm_w, router_w, w1, b1, w2, b2):
    B, T, D = x.shape
    H, HD = 16, 64
    E = w1.shape[0]
    x2 = x.reshape(T, D)

    wqkv = jnp.concatenate([wq, wk, wv], axis=1)
    qkv = _norm_mm(x2, attn_norm_w, wqkv)
    q = qkv[:, :D].reshape(T, H, HD).transpose(1, 0, 2)
    kt = qkv[:, D:2 * D].reshape(T, H, HD).transpose(1, 2, 0)
    v = qkv[:, 2 * D:].reshape(T, H, HD).transpose(1, 0, 2)

    o = _attn(q, kt, v)
    attn2 = o.transpose(1, 0, 2).reshape(T, D)

    x1, h2 = _out_norm(attn2, wo, x2, mlp_norm_w)

    rw_pad = jnp.pad(router_w, ((0, 0), (0, 128 - E)))
    wts, bal = _router(h2, rw_pad, E)

    out = _moe(h2, wts, x1, w1, b1, w2, b2)
    return out.reshape(B, T, D), bal[0, 0]


# speedup vs baseline: 1.1943x; 1.1943x over previous
import functools

import jax
import jax.numpy as jnp
from jax.experimental import pallas as pl

EPS = 1e-6


def _dot(a, b):
    return jnp.dot(a, b, preferred_element_type=jnp.float32)


def _norm_mm_body(x_ref, nw_ref, w_ref, o_ref):
    xb = x_ref[...]
    v = jnp.mean(xb * xb, axis=1, keepdims=True)
    normed = xb * jax.lax.rsqrt(v + EPS) * nw_ref[...]
    o_ref[...] = _dot(normed, w_ref[...])


def _norm_mm(x, nw, w, bm=512, bn=1024):
    T, D = x.shape
    N = w.shape[1]
    return pl.pallas_call(
        _norm_mm_body,
        grid=(T // bm, N // bn),
        in_specs=[
            pl.BlockSpec((bm, D), lambda i, j: (i, 0)),
            pl.BlockSpec((1, D), lambda i, j: (0, 0)),
            pl.BlockSpec((D, bn), lambda i, j: (0, j)),
        ],
        out_specs=pl.BlockSpec((bm, bn), lambda i, j: (i, j)),
        out_shape=jax.ShapeDtypeStruct((T, N), jnp.float32),
    )(x, nw.reshape(1, D), w)


def _attn_body(q_ref, kt_ref, v_ref, o_ref, *, bq, T, scale):
    qb = pl.program_id(1)
    s = _dot(q_ref[0], kt_ref[0]) * scale
    col = jax.lax.broadcasted_iota(jnp.int32, (bq, T), 1)
    row = qb * bq + jax.lax.broadcasted_iota(jnp.int32, (bq, T), 0)
    s = jnp.where(col <= row, s, jnp.float32(-1e9))
    m = jnp.max(s, axis=1, keepdims=True)
    p = jnp.exp(s - m)
    a = p / jnp.sum(p, axis=1, keepdims=True)
    o_ref[0] = _dot(a, v_ref[0])


def _attn(q, kt, v, bq=256):
    H, T, HD = q.shape
    body = functools.partial(_attn_body, bq=bq, T=T,
                             scale=1.0 / (HD ** 0.5))
    return pl.pallas_call(
        body,
        grid=(H, T // bq),
        in_specs=[
            pl.BlockSpec((1, bq, HD), lambda h, i: (h, i, 0)),
            pl.BlockSpec((1, HD, T), lambda h, i: (h, 0, 0)),
            pl.BlockSpec((1, T, HD), lambda h, i: (h, 0, 0)),
        ],
        out_specs=pl.BlockSpec((1, bq, HD), lambda h, i: (h, i, 0)),
        out_shape=jax.ShapeDtypeStruct((H, T, HD), jnp.float32),
    )(q, kt, v)


def _out_norm_body(a_ref, wo_ref, x_ref, nw_ref, x1_ref, h2_ref):
    x1 = _dot(a_ref[...], wo_ref[...]) + x_ref[...]
    x1_ref[...] = x1
    v = jnp.mean(x1 * x1, axis=1, keepdims=True)
    h2_ref[...] = x1 * jax.lax.rsqrt(v + EPS) * nw_ref[...]


def _out_norm(attn2, wo, x, nw, bm=512):
    T, D = x.shape
    return pl.pallas_call(
        _out_norm_body,
        grid=(T // bm,),
        in_specs=[
            pl.BlockSpec((bm, D), lambda i: (i, 0)),
            pl.BlockSpec((D, D), lambda i: (0, 0)),
            pl.BlockSpec((bm, D), lambda i: (i, 0)),
            pl.BlockSpec((1, D), lambda i: (0, 0)),
        ],
        out_specs=[
            pl.BlockSpec((bm, D), lambda i: (i, 0)),
            pl.BlockSpec((bm, D), lambda i: (i, 0)),
        ],
        out_shape=[
            jax.ShapeDtypeStruct((T, D), jnp.float32),
            jax.ShapeDtypeStruct((T, D), jnp.float32),
        ],
    )(attn2, wo, x, nw.reshape(1, D))


def _router_body(h2_ref, rw_ref, w_ref, bal_ref, *, T, E):
    logits = _dot(h2_ref[...], rw_ref[...])
    lane = jax.lax.broadcasted_iota(jnp.int32, logits.shape, 1)
    valid = lane < E
    logits = jnp.where(valid, logits, jnp.float32(-1e30))
    m = jnp.max(logits, axis=1, keepdims=True)
    p = jnp.exp(logits - m)
    p = jnp.where(valid, p, 0.0)
    psum = jnp.sum(p, axis=1, keepdims=True)
    probs = p / psum
    pmin = jnp.min(jnp.where(valid, probs, jnp.float32(2.0)),
                   axis=1, keepdims=True)
    p1 = jnp.sum(jnp.where(lane == 1, probs, 0.0), axis=1, keepdims=True)
    p2 = jnp.sum(jnp.where(lane == 2, probs, 0.0), axis=1, keepdims=True)
    excl = jnp.where(p2 == pmin, 2, jnp.where(p1 == pmin, 1, 0))
    keep = valid & (lane != excl)
    denom = jnp.sum(probs, axis=1, keepdims=True) - pmin
    w_ref[...] = jnp.where(keep, probs, 0.0) / denom
    ft = jnp.sum(keep.astype(jnp.float32), axis=0, keepdims=True) / (T * 2.0)
    fp = jnp.sum(probs, axis=0, keepdims=True) / T
    bal = E * jnp.sum(ft * fp, axis=1, keepdims=True)
    bal_ref[...] = jnp.broadcast_to(bal, bal_ref.shape)


def _router(h2, rw_pad, E):
    T, D = h2.shape
    body = functools.partial(_router_body, T=T, E=E)
    return pl.pallas_call(
        body,
        grid=(1,),
        in_specs=[
            pl.BlockSpec((T, D), lambda i: (0, 0)),
            pl.BlockSpec((D, 128), lambda i: (0, 0)),
        ],
        out_specs=[
            pl.BlockSpec((T, 128), lambda i: (0, 0)),
            pl.BlockSpec((8, 128), lambda i: (0, 0)),
        ],
        out_shape=[
            jax.ShapeDtypeStruct((T, 128), jnp.float32),
            jax.ShapeDtypeStruct((8, 128), jnp.float32),
        ],
    )(h2, rw_pad)


def _moe_body(h2_ref, wts_ref, x1_ref, w1_ref, b1_ref, w2_ref, b2_ref, o_ref):
    e = pl.program_id(1)
    f = pl.program_id(2)
    lane = jax.lax.broadcasted_iota(jnp.int32, wts_ref.shape, 1)
    we = jnp.sum(jnp.where(lane == e, wts_ref[...], 0.0),
                 axis=1, keepdims=True)

    @pl.when((e == 0) & (f == 0))
    def _init():
        o_ref[...] = x1_ref[...]

    @pl.when(f == 0)
    def _bias2():
        o_ref[...] += we * b2_ref[0]

    xb = h2_ref[...].astype(jnp.bfloat16)
    h = jax.nn.gelu(_dot(xb, w1_ref[0]) + b1_ref[0])
    hb = (we * h).astype(jnp.bfloat16)
    o_ref[...] += _dot(hb, w2_ref[0])


def _moe(h2, wts, x1, w1, b1, w2, b2, bm=512, bf=512):
    T, D = h2.shape
    E, _, F = w1.shape
    return pl.pallas_call(
        _moe_body,
        grid=(T // bm, E, F // bf),
        in_specs=[
            pl.BlockSpec((bm, D), lambda t, e, f: (t, 0)),
            pl.BlockSpec((bm, 128), lambda t, e, f: (t, 0)),
            pl.BlockSpec((bm, D), lambda t, e, f: (t, 0)),
            pl.BlockSpec((1, D, bf), lambda t, e, f: (e, 0, f)),
            pl.BlockSpec((1, 1, bf), lambda t, e, f: (e, 0, f)),
            pl.BlockSpec((1, bf, D), lambda t, e, f: (e, f, 0)),
            pl.BlockSpec((1, 1, D), lambda t, e, f: (e, 0, 0)),
        ],
        out_specs=pl.BlockSpec((bm, D), lambda t, e, f: (t, 0)),
        out_shape=jax.ShapeDtypeStruct((T, D), jnp.float32),
    )(h2, wts, x1, w1.astype(jnp.bfloat16), b1.reshape(E, 1, F),
      w2.astype(jnp.bfloat16), b2.reshape(E, 1, D))


def kernel(x, attn_norm_w, wq, wk, wv, wo, mlp_norm_w, router_w, w1, b1, w2, b2):
    B, T, D = x.shape
    H, HD = 16, 64
    E = w1.shape[0]
    x2 = x.reshape(T, D)

    wqkv = jnp.concatenate([wq, wk, wv], axis=1)
    qkv = _norm_mm(x2, attn_norm_w, wqkv)
    q = qkv[:, :D].reshape(T, H, HD).transpose(1, 0, 2)
    kt = qkv[:, D:2 * D].reshape(T, H, HD).transpose(1, 2, 0)
    v = qkv[:, 2 * D:].reshape(T, H, HD).transpose(1, 0, 2)

    o = _attn(q, kt, v)
    attn2 = o.transpose(1, 0, 2).reshape(T, D)

    x1, h2 = _out_norm(attn2, wo, x2, mlp_norm_w)

    rw_pad = jnp.pad(router_w, ((0, 0), (0, 128 - E)))
    wts, bal = _router(h2, rw_pad, E)

    out = _moe(h2, wts, x1, w1, b1, w2, b2)
    return out.reshape(B, T, D), bal[0, 0]


# speedup vs baseline: 1.2104x; 1.0135x over previous
import functools

import jax
import jax.numpy as jnp
from jax.experimental import pallas as pl
from jax.experimental.pallas import tpu as pltpu

EPS = 1e-6
BF = jnp.bfloat16


def _dot(a, b):
    return jnp.dot(a, b, preferred_element_type=jnp.float32)


def _norm_mm_body(x_ref, nw_ref, w_ref, o_ref):
    xb = x_ref[...]
    v = jnp.mean(xb * xb, axis=1, keepdims=True)
    normed = xb * jax.lax.rsqrt(v + EPS) * nw_ref[...]
    o_ref[...] = _dot(normed.astype(BF), w_ref[...])


def _norm_mm(x, nw, w, bm=512, bn=1024):
    T, D = x.shape
    N = w.shape[1]
    return pl.pallas_call(
        _norm_mm_body,
        grid=(T // bm, N // bn),
        in_specs=[
            pl.BlockSpec((bm, D), lambda i, j: (i, 0)),
            pl.BlockSpec((1, D), lambda i, j: (0, 0)),
            pl.BlockSpec((D, bn), lambda i, j: (0, j)),
        ],
        out_specs=pl.BlockSpec((bm, bn), lambda i, j: (i, j)),
        out_shape=jax.ShapeDtypeStruct((T, N), jnp.float32),
    )(x, nw.reshape(1, D), w.astype(BF))


def _attn_body(q_ref, kt_ref, v_ref, o_ref, s_scr, *, bq, T, scale):
    qb = pl.program_id(1)
    HD = q_ref.shape[2]
    q = q_ref[0]

    s_scr[...] = jnp.full(s_scr.shape, -1e9, jnp.float32)

    def loop1(j, carry):
        kj = kt_ref[0, :, pl.ds(j * bq, bq)]
        s = _dot(q, kj) * scale
        col = j * bq + jax.lax.broadcasted_iota(jnp.int32, (bq, bq), 1)
        row = qb * bq + jax.lax.broadcasted_iota(jnp.int32, (bq, bq), 0)
        s_scr[:, pl.ds(j * bq, bq)] = jnp.where(col <= row, s,
                                                jnp.float32(-1e9))
        return carry

    jax.lax.fori_loop(0, qb + 1, loop1, 0)

    s = s_scr[...]
    m = jnp.max(s, axis=1, keepdims=True)
    p = jnp.exp(s - m)
    l = jnp.sum(p, axis=1, keepdims=True)
    o_ref[0] = _dot(p.astype(BF), v_ref[0]) / l


def _attn(q, kt, v, bq=256):
    H, T, HD = q.shape
    body = functools.partial(_attn_body, bq=bq, T=T,
                             scale=1.0 / (HD ** 0.5))
    return pl.pallas_call(
        body,
        grid=(H, T // bq),
        in_specs=[
            pl.BlockSpec((1, bq, HD), lambda h, i: (h, i, 0)),
            pl.BlockSpec((1, HD, T), lambda h, i: (h, 0, 0)),
            pl.BlockSpec((1, T, HD), lambda h, i: (h, 0, 0)),
        ],
        out_specs=pl.BlockSpec((1, bq, HD), lambda h, i: (h, i, 0)),
        out_shape=jax.ShapeDtypeStruct((H, T, HD), jnp.float32),
        scratch_shapes=[pltpu.VMEM((bq, T), jnp.float32)],
    )(q, kt, v)


def _out_norm_body(a_ref, wo_ref, x_ref, nw_ref, x1_ref, h2_ref):
    x1 = _dot(a_ref[...].astype(BF), wo_ref[...]) + x_ref[...]
    x1_ref[...] = x1
    v = jnp.mean(x1 * x1, axis=1, keepdims=True)
    h2_ref[...] = x1 * jax.lax.rsqrt(v + EPS) * nw_ref[...]


def _out_norm(attn2, wo, x, nw, bm=512):
    T, D = x.shape
    return pl.pallas_call(
        _out_norm_body,
        grid=(T // bm,),
        in_specs=[
            pl.BlockSpec((bm, D), lambda i: (i, 0)),
            pl.BlockSpec((D, D), lambda i: (0, 0)),
            pl.BlockSpec((bm, D), lambda i: (i, 0)),
            pl.BlockSpec((1, D), lambda i: (0, 0)),
        ],
        out_specs=[
            pl.BlockSpec((bm, D), lambda i: (i, 0)),
            pl.BlockSpec((bm, D), lambda i: (i, 0)),
        ],
        out_shape=[
            jax.ShapeDtypeStruct((T, D), jnp.float32),
            jax.ShapeDtypeStruct((T, D), jnp.float32),
        ],
    )(attn2, wo.astype(BF), x, nw.reshape(1, D))


def _router_body(h2_ref, rw_ref, w_ref, bal_ref, *, T, E):
    logits = _dot(h2_ref[...].astype(BF), rw_ref[...])
    lane = jax.lax.broadcasted_iota(jnp.int32, logits.shape, 1)
    valid = lane < E
    logits = jnp.where(valid, logits, jnp.float32(-1e30))
    m = jnp.max(logits, axis=1, keepdims=True)
    p = jnp.exp(logits - m)
    p = jnp.where(valid, p, 0.0)
    psum = jnp.sum(p, axis=1, keepdims=True)
    probs = p / psum
    pmin = jnp.min(jnp.where(valid, probs, jnp.float32(2.0)),
                   axis=1, keepdims=True)
    p1 = jnp.sum(jnp.where(lane == 1, probs, 0.0), axis=1, keepdims=True)
    p2 = jnp.sum(jnp.where(lane == 2, probs, 0.0), axis=1, keepdims=True)
    excl = jnp.where(p2 == pmin, 2, jnp.where(p1 == pmin, 1, 0))
    keep = valid & (lane != excl)
    denom = jnp.sum(probs, axis=1, keepdims=True) - pmin
    w_ref[...] = jnp.where(keep, probs, 0.0) / denom
    ft = jnp.sum(keep.astype(jnp.float32), axis=0, keepdims=True) / (T * 2.0)
    fp = jnp.sum(probs, axis=0, keepdims=True) / T
    bal = E * jnp.sum(ft * fp, axis=1, keepdims=True)
    bal_ref[...] = jnp.broadcast_to(bal, bal_ref.shape)


def _router(h2, rw_pad, E):
    T, D = h2.shape
    body = functools.partial(_router_body, T=T, E=E)
    return pl.pallas_call(
        body,
        grid=(1,),
        in_specs=[
            pl.BlockSpec((T, D), lambda i: (0, 0)),
            pl.BlockSpec((D, 128), lambda i: (0, 0)),
        ],
        out_specs=[
            pl.BlockSpec((T, 128), lambda i: (0, 0)),
            pl.BlockSpec((8, 128), lambda i: (0, 0)),
        ],
        out_shape=[
            jax.ShapeDtypeStruct((T, 128), jnp.float32),
            jax.ShapeDtypeStruct((8, 128), jnp.float32),
        ],
    )(h2, rw_pad.astype(BF))


def _moe_body(h2_ref, wts_ref, x1_ref, w1_ref, b1_ref, w2_ref, b2_ref, o_ref):
    e = pl.program_id(1)
    f = pl.program_id(2)
    lane = jax.lax.broadcasted_iota(jnp.int32, wts_ref.shape, 1)
    we = jnp.sum(jnp.where(lane == e, wts_ref[...], 0.0),
                 axis=1, keepdims=True)

    @pl.when((e == 0) & (f == 0))
    def _init():
        o_ref[...] = x1_ref[...]

    @pl.when(f == 0)
    def _bias2():
        o_ref[...] += we * b2_ref[0]

    xb = h2_ref[...].astype(BF)
    h = jax.nn.gelu(_dot(xb, w1_ref[0]) + b1_ref[0])
    o_ref[...] += we * _dot(h.astype(BF), w2_ref[0])


def _moe(h2, wts, x1, w1, b1, w2, b2, bm=512, bf=512):
    T, D = h2.shape
    E, _, F = w1.shape
    return pl.pallas_call(
        _moe_body,
        grid=(T // bm, E, F // bf),
        in_specs=[
            pl.BlockSpec((bm, D), lambda t, e, f: (t, 0)),
            pl.BlockSpec((bm, 128), lambda t, e, f: (t, 0)),
            pl.BlockSpec((bm, D), lambda t, e, f: (t, 0)),
            pl.BlockSpec((1, D, bf), lambda t, e, f: (e, 0, f)),
            pl.BlockSpec((1, 1, bf), lambda t, e, f: (e, 0, f)),
            pl.BlockSpec((1, bf, D), lambda t, e, f: (e, f, 0)),
            pl.BlockSpec((1, 1, D), lambda t, e, f: (e, 0, 0)),
        ],
        out_specs=pl.BlockSpec((bm, D), lambda t, e, f: (t, 0)),
        out_shape=jax.ShapeDtypeStruct((T, D), jnp.float32),
    )(h2, wts, x1, w1.astype(BF), b1.reshape(E, 1, F),
      w2.astype(BF), b2.reshape(E, 1, D))


def kernel(x, attn_norm_w, wq, wk, wv, wo, mlp_norm_w, router_w, w1, b1, w2, b2):
    B, T, D = x.shape
    H, HD = 16, 64
    E = w1.shape[0]
    x2 = x.reshape(T, D)

    wqkv = jnp.concatenate([wq, wk, wv], axis=1)
    qkv = _norm_mm(x2, attn_norm_w, wqkv)
    q = qkv[:, :D].reshape(T, H, HD).transpose(1, 0, 2).astype(BF)
    kt = qkv[:, D:2 * D].reshape(T, H, HD).transpose(1, 2, 0).astype(BF)
    v = qkv[:, 2 * D:].reshape(T, H, HD).transpose(1, 0, 2).astype(BF)

    o = _attn(q, kt, v)
    attn2 = o.transpose(1, 0, 2).reshape(T, D)

    x1, h2 = _out_norm(attn2, wo, x2, mlp_norm_w)

    rw_pad = jnp.pad(router_w, ((0, 0), (0, 128 - E)))
    wts, bal = _router(h2, rw_pad, E)

    out = _moe(h2, wts, x1, w1, b1, w2, b2)
    return out.reshape(B, T, D), bal[0, 0]
